# Initial kernel scaffold; baseline (speedup 1.0000x reference)
import jax
import jax.numpy as jnp
from jax.experimental import pallas as pl


def kernel(x_g, edge_index_g, x_b, edge_index_b, W1g, b1g, W2g, b2g, W1b, b1b, W2b, b2b, Wint):
    raise NotImplementedError("write your pallas kernel here")


# speedup vs baseline: 4.8680x; 4.8680x over previous
import functools

import jax
import jax.numpy as jnp
from jax import lax
from jax.experimental import pallas as pl
from jax.experimental.pallas import tpu as pltpu
from jax.experimental.pallas import tpu_sc as plsc

N_AG = 10000
N_AB = 2000
D_IN = 128
D_HID = 128
D_OUT = 64

NPG = 10240
NPB = 2048
EPG = 163840
EPB = 32768

NC = 2
NS = 16
NW = NC * NS
K = 128
BLK = 256


@functools.lru_cache(maxsize=None)
def _make_segsum(n_pad, d, e_pad):
  e_per_w = e_pad // NW
  chunks = e_per_w // K
  rpt = n_pad // NS
  mesh = plsc.VectorSubcoreMesh(core_axis_name="c", subcore_axis_name="s")

  @functools.partial(
      pl.kernel,
      mesh=mesh,
      out_type=jax.ShapeDtypeStruct((NC, n_pad, d), jnp.float32),
      scratch_types=[
          pltpu.VMEM((K,), jnp.int32),
          pltpu.VMEM((K,), jnp.int32),
          pltpu.VMEM((K, d), jnp.float32),
          pltpu.VMEM_SHARED((n_pad, d), jnp.float32),
          pltpu.SemaphoreType.DMA,
      ],
      compiler_params=pltpu.CompilerParams(use_tc_tiling_on_sc=False),
  )
  def seg(src_hbm, dst_hbm, tab_hbm, zero_hbm, out_hbm,
          src_v, dst_v, rows_v, acc_sh, sem):
    c = lax.axis_index("c")
    s = lax.axis_index("s")
    wid = s * NC + c
    r0 = s * rpt
    pltpu.sync_copy(zero_hbm.at[pl.ds(r0, rpt)], acc_sh.at[pl.ds(r0, rpt)])
    plsc.subcore_barrier()

    e0 = wid * e_per_w

    def body(i, carry):
      off = e0 + i * K
      pltpu.sync_copy(src_hbm.at[pl.ds(off, K)], src_v)
      pltpu.sync_copy(dst_hbm.at[pl.ds(off, K)], dst_v)
      pltpu.async_copy(tab_hbm.at[src_v], rows_v, sem).wait()
      pltpu.sync_copy(rows_v, acc_sh.at[dst_v], add=True)
      return carry

    lax.fori_loop(0, chunks, body, 0)
    plsc.subcore_barrier()
    pltpu.sync_copy(acc_sh.at[pl.ds(r0, rpt)], out_hbm.at[c, pl.ds(r0, rpt)])

  return seg


@functools.lru_cache(maxsize=None)
def _make_degsum(n_pad, e_pad):
  d = 16
  e_per_w = e_pad // NW
  chunks = e_per_w // K
  rpt = n_pad // NS
  mesh = plsc.VectorSubcoreMesh(core_axis_name="c", subcore_axis_name="s")

  @functools.partial(
      pl.kernel,
      mesh=mesh,
      out_type=jax.ShapeDtypeStruct((NC, n_pad, d), jnp.float32),
      scratch_types=[
          pltpu.VMEM((K,), jnp.int32),
          pltpu.VMEM((K, d), jnp.float32),
          pltpu.VMEM_SHARED((n_pad, d), jnp.float32),
      ],
      compiler_params=pltpu.CompilerParams(use_tc_tiling_on_sc=False),
  )
  def deg(dst_hbm, ones_hbm, zero_hbm, out_hbm, dst_v, rows_v, acc_sh):
    c = lax.axis_index("c")
    s = lax.axis_index("s")
    wid = s * NC + c
    r0 = s * rpt
    pltpu.sync_copy(zero_hbm.at[pl.ds(r0, rpt)], acc_sh.at[pl.ds(r0, rpt)])
    pltpu.sync_copy(ones_hbm, rows_v)
    plsc.subcore_barrier()

    e0 = wid * e_per_w

    def body(i, carry):
      off = e0 + i * K
      pltpu.sync_copy(dst_hbm.at[pl.ds(off, K)], dst_v)
      pltpu.sync_copy(rows_v, acc_sh.at[dst_v], add=True)
      return carry

    lax.fori_loop(0, chunks, body, 0)
    plsc.subcore_barrier()
    pltpu.sync_copy(acc_sh.at[pl.ds(r0, rpt)], out_hbm.at[c, pl.ds(r0, rpt)])

  return deg


def _dis_from_parts(dp):
  deg = dp[0, :, 0] + dp[1, :, 0] + 1.0
  return lax.rsqrt(deg)


def _mm_scale_body(x_ref, w_ref, dp_ref, out_ref):
  dis = _dis_from_parts(dp_ref[...])
  h = jnp.dot(x_ref[...], w_ref[...], preferred_element_type=jnp.float32)
  out_ref[...] = h * dis[:, None]


def _mm_scale(x, w, dp, n_pad, d_in, d_out):
  return pl.pallas_call(
      _mm_scale_body,
      grid=(n_pad // BLK,),
      in_specs=[
          pl.BlockSpec((BLK, d_in), lambda i: (i, 0)),
          pl.BlockSpec((d_in, d_out), lambda i: (0, 0)),
          pl.BlockSpec((NC, BLK, 16), lambda i: (0, i, 0)),
      ],
      out_specs=pl.BlockSpec((BLK, d_out), lambda i: (i, 0)),
      out_shape=jax.ShapeDtypeStruct((n_pad, d_out), jnp.float32),
  )(x, w, dp)


def _combine_mm_body(acc_ref, hs_ref, dp_ref, b_ref, w_ref, out_ref):
  dis = _dis_from_parts(dp_ref[...])
  s = acc_ref[0] + acc_ref[1] + hs_ref[...]
  h = jnp.maximum(s * dis[:, None] + b_ref[...], 0.0)
  h2 = jnp.dot(h, w_ref[...], preferred_element_type=jnp.float32)
  out_ref[...] = h2 * dis[:, None]


def _combine_mm(acc, hs, dp, b, w, n_pad, d_in, d_out):
  return pl.pallas_call(
      _combine_mm_body,
      grid=(n_pad // BLK,),
      in_specs=[
          pl.BlockSpec((NC, BLK, d_in), lambda i: (0, i, 0)),
          pl.BlockSpec((BLK, d_in), lambda i: (i, 0)),
          pl.BlockSpec((NC, BLK, 16), lambda i: (0, i, 0)),
          pl.BlockSpec((1, d_in), lambda i: (0, 0)),
          pl.BlockSpec((d_in, d_out), lambda i: (0, 0)),
      ],
      out_specs=pl.BlockSpec((BLK, d_out), lambda i: (i, 0)),
      out_shape=jax.ShapeDtypeStruct((n_pad, d_out), jnp.float32),
  )(acc, hs, dp, b, w)


def _combine_body(acc_ref, hs_ref, dp_ref, b_ref, out_ref):
  dis = _dis_from_parts(dp_ref[...])
  s = acc_ref[0] + acc_ref[1] + hs_ref[...]
  out_ref[...] = jnp.maximum(s * dis[:, None] + b_ref[...], 0.0)


def _combine(acc, hs, dp, b, n_pad, d):
  return pl.pallas_call(
      _combine_body,
      grid=(n_pad // BLK,),
      in_specs=[
          pl.BlockSpec((NC, BLK, d), lambda i: (0, i, 0)),
          pl.BlockSpec((BLK, d), lambda i: (i, 0)),
          pl.BlockSpec((NC, BLK, 16), lambda i: (0, i, 0)),
          pl.BlockSpec((1, d), lambda i: (0, 0)),
      ],
      out_specs=pl.BlockSpec((BLK, d), lambda i: (i, 0)),
      out_shape=jax.ShapeDtypeStruct((n_pad, d), jnp.float32),
  )(acc, hs, dp, b)


def _decoder_body(ag_ref, wint_ref, ab_ref, ip_ref, epi_ref):
  a2 = jnp.dot(ag_ref[...], wint_ref[...], preferred_element_type=jnp.float32)
  logits = lax.dot_general(
      a2, ab_ref[...],
      dimension_numbers=(((1,), (1,)), ((), ())),
      preferred_element_type=jnp.float32,
  )
  ip_ref[...] = jax.nn.sigmoid(logits)
  cols = lax.broadcasted_iota(jnp.int32, logits.shape, 1)
  masked = jnp.where(cols < N_AB, logits, -1e30)
  epi_ref[...] = jax.nn.sigmoid(jnp.max(masked, axis=1))[:, None]


def _decoder(ag, wint, ab):
  return pl.pallas_call(
      _decoder_body,
      grid=(NPG // BLK,),
      in_specs=[
          pl.BlockSpec((BLK, D_OUT), lambda i: (i, 0)),
          pl.BlockSpec((D_OUT, D_OUT), lambda i: (0, 0)),
          pl.BlockSpec((NPB, D_OUT), lambda i: (0, 0)),
      ],
      out_specs=[
          pl.BlockSpec((BLK, NPB), lambda i: (i, 0)),
          pl.BlockSpec((BLK, 1), lambda i: (i, 0)),
      ],
      out_shape=[
          jax.ShapeDtypeStruct((NPG, NPB), jnp.float32),
          jax.ShapeDtypeStruct((NPG, 1), jnp.float32),
      ],
  )(ag, wint, ab)


def _pad_edges(edge_index, n, e_pad):
  src = edge_index[0].astype(jnp.int32)
  dst = edge_index[1].astype(jnp.int32)
  e = src.shape[0]
  src_p = jnp.full((e_pad,), n, jnp.int32).at[:e].set(src)
  dst_p = jnp.full((e_pad,), n, jnp.int32).at[:e].set(dst)
  return src_p, dst_p


def _encode(x, src, dst, w1, b1, w2, b2, n, n_pad, e_pad):
  x_p = jnp.zeros((n_pad, D_IN), jnp.float32).at[:n].set(x)
  ones16 = jnp.ones((K, 16), jnp.float32)
  z16 = jnp.zeros((n_pad, 16), jnp.float32)
  z_hid = jnp.zeros((n_pad, D_HID), jnp.float32)
  z_out = jnp.zeros((n_pad, D_OUT), jnp.float32)

  dp = _make_degsum(n_pad, e_pad)(dst, ones16, z16)
  hs1 = _mm_scale(x_p, w1, dp, n_pad, D_IN, D_HID)
  acc1 = _make_segsum(n_pad, D_HID, e_pad)(src, dst, hs1, z_hid)
  hs2 = _combine_mm(acc1, hs1, dp, b1.reshape(1, -1), w2, n_pad, D_HID, D_OUT)
  acc2 = _make_segsum(n_pad, D_OUT, e_pad)(src, dst, hs2, z_out)
  emb = _combine(acc2, hs2, dp, b2.reshape(1, -1), n_pad, D_OUT)
  return emb


@jax.jit
def kernel(x_g, edge_index_g, x_b, edge_index_b,
           W1g, b1g, W2g, b2g, W1b, b1b, W2b, b2b, Wint):
  src_g, dst_g = _pad_edges(edge_index_g, N_AG, EPG)
  src_b, dst_b = _pad_edges(edge_index_b, N_AB, EPB)

  ag_emb = _encode(x_g, src_g, dst_g, W1g, b1g, W2g, b2g, N_AG, NPG, EPG)
  ab_emb = _encode(x_b, src_b, dst_b, W1b, b1b, W2b, b2b, N_AB, NPB, EPB)

  ip_p, epi_p = _decoder(ag_emb, Wint, ab_emb)

  return (ag_emb[:N_AG], ab_emb[:N_AB],
          ip_p[:N_AG, :N_AB], epi_p[:N_AG, 0])
